# baseline (speedup 1.0000x reference)
import functools

import jax
import jax.numpy as jnp
from jax import lax
from jax.experimental import pallas as pl
from jax.experimental.pallas import tpu as pltpu
from jax.experimental.pallas import tpu_sc as plsc

NC = 2
NS = 16
NW = NC * NS
C = 64
DIM = 300
RPT = 3


@functools.lru_cache(maxsize=None)
def _make_lookup(n_tokens: int):
    b_per_w = n_tokens // NW
    n_chunks = b_per_w // C
    idx_rows = b_per_w // 128
    mesh = plsc.VectorSubcoreMesh(core_axis_name="c", subcore_axis_name="s")

    @functools.partial(
        pl.kernel,
        mesh=mesh,
        out_type=jax.ShapeDtypeStruct((n_tokens * DIM,), jnp.float32),
        scratch_types=[
            pltpu.VMEM((b_per_w,), jnp.int32),
            pltpu.VMEM((RPT, 128), jnp.int32),
            pltpu.VMEM((RPT * C, 128), jnp.float32),
            pltpu.VMEM((C * DIM + 16,), jnp.float32),
            pltpu.SemaphoreType.DMA,
        ],
    )
    def lookup(idx_hbm, table_hbm, out_hbm, idx_v, idx3_v, stage_v, packed_v,
               g_sem):
        w = lax.axis_index("s") * NC + lax.axis_index("c")
        pltpu.sync_copy(idx_hbm.at[pl.ds(w * b_per_w, b_per_w)], idx_v)

        def chunk_body(g, carry):
            for u in range(C // 16):
                v16 = idx_v[pl.ds(g * C + 16 * u, 16)]
                b3 = v16 * RPT
                for j in range(RPT):
                    idx3_v[j, pl.ds(16 * u, 16)] = b3 + j
            cps = [
                pltpu.async_copy(
                    table_hbm.at[idx3_v.at[j, pl.ds(0, C)]],
                    stage_v.at[pl.ds(C * j, C)],
                    g_sem,
                )
                for j in range(RPT)
            ]
            for cp in cps:
                cp.wait()

            def pack_group(m, c2):
                tb = 16 * m
                pb = 16 * m * DIM
                for tp in range(16):
                    for k in range(19):
                        if k < 8:
                            row, col = tb + tp, 16 * k
                        elif k < 16:
                            row, col = C + tb + tp, 16 * (k - 8)
                        elif k < 18:
                            row, col = 2 * C + tb + tp, 16 * (k - 16)
                        else:
                            row, col = 2 * C + tb + tp, 28
                        dst = pb + DIM * tp + (16 * k if k < 18 else 284)
                        packed_v[pl.ds(dst, 16)] = stage_v[row, pl.ds(col, 16)]
                return c2

            lax.fori_loop(0, C // 16, pack_group, 0)
            off = (w * b_per_w + g * C) * DIM
            pltpu.sync_copy(
                packed_v.at[pl.ds(0, C * DIM)],
                out_hbm.at[pl.ds(off, C * DIM)],
            )
            return carry

        lax.fori_loop(0, n_chunks, chunk_body, 0)

    return lookup


def kernel(input_ids, emb_weight):
    b, l = input_ids.shape
    vocab, dim = emb_weight.shape
    n = b * l
    idx = input_ids.reshape(n).astype(jnp.int32)
    table = jnp.pad(emb_weight, ((0, 0), (0, RPT * 128 - dim)))
    table = table.reshape(vocab * RPT, 128)
    out = _make_lookup(n)(idx, table)
    return out.reshape(b, l, dim)


# speedup vs baseline: 1.9218x; 1.9218x over previous
import functools

import jax
import jax.numpy as jnp
from jax import lax
from jax.experimental import pallas as pl
from jax.experimental.pallas import tpu as pltpu
from jax.experimental.pallas import tpu_sc as plsc

NC = 2
NS = 16
NW = NC * NS
CHUNK = 128
DPAD = 384


@functools.lru_cache(maxsize=None)
def _make_lookup(n_tokens: int):
    assert n_tokens % (NW * CHUNK) == 0
    b_per_w = n_tokens // NW
    n_chunks = b_per_w // CHUNK
    mesh = plsc.VectorSubcoreMesh(core_axis_name="c", subcore_axis_name="s")

    @functools.partial(
        pl.kernel,
        mesh=mesh,
        out_type=jax.ShapeDtypeStruct((n_tokens, DPAD), jnp.float32),
        scratch_types=[
            pltpu.VMEM((n_chunks, CHUNK), jnp.int32),
            pltpu.VMEM((CHUNK, DPAD), jnp.float32),
            pltpu.SemaphoreType.DMA,
        ],
    )
    def lookup(idx_hbm, table_hbm, out_hbm, idx_v, rows_v, g_sem):
        wid = lax.axis_index("s") * NC + lax.axis_index("c")
        base = wid * b_per_w
        pltpu.sync_copy(idx_hbm.at[wid], idx_v)

        def body(g, carry):
            pltpu.async_copy(table_hbm.at[idx_v.at[g]], rows_v, g_sem).wait()
            pltpu.sync_copy(rows_v, out_hbm.at[pl.ds(base + g * CHUNK, CHUNK)])
            return carry

        lax.fori_loop(0, n_chunks, body, 0)

    return lookup


def kernel(input_ids, emb_weight):
    b, l = input_ids.shape
    vocab, dim = emb_weight.shape
    n = b * l
    idx = input_ids.reshape(NW, n // (NW * CHUNK), CHUNK).astype(jnp.int32)
    table = jnp.pad(emb_weight, ((0, 0), (0, DPAD - dim)))
    out = _make_lookup(n)(idx, table)
    one = (1 + 0 * input_ids[0, 0]).astype(jnp.float32)
    return (out[:, :dim] * one).reshape(b, l, dim)
